# baseline (speedup 1.0000x reference)
import functools

import jax
import jax.numpy as jnp
from jax import lax
from jax.experimental import pallas as pl
from jax.experimental.pallas import tpu as pltpu
from jax.experimental.pallas import tpu_sc as plsc

OUT_F = 8192
IN_F = 8192
K = 16
T = 512
L = 16
NC = 2
NS = 16
NW = NC * NS
O_PER_W = OUT_F // NW
CHUNKS = T // L
G = 4
GK = G * K
NGRP = O_PER_W // G
CUNROLL = 8


def _bcast_lane(vec, lane):
    idx = jnp.broadcast_to(jnp.asarray(lane, jnp.int32), (L,))
    return vec.at[idx].get(mode="promise_in_bounds")


def _sc_body(xT_hbm, idx_hbm, w_hbm, b_hbm, out_hbm,
             idx_v, w_v, b_v, rows2, obuf2,
             gsem0, gsem1, osem0, osem1):
    wid = lax.axis_index("s") * NC + lax.axis_index("c")
    base = wid * O_PER_W

    pltpu.sync_copy(idx_hbm.at[pl.ds(base * K, O_PER_W * K)], idx_v)
    pltpu.sync_copy(w_hbm.at[pl.ds(base, O_PER_W)], w_v)
    pltpu.sync_copy(b_hbm.at[pl.ds(wid * (O_PER_W // L), O_PER_W // L)], b_v)

    gsems = (gsem0, gsem1)
    osems = (osem0, osem1)

    def gather_desc(g, buf, sem):
        return pltpu.make_async_copy(
            xT_hbm.at[idx_v.at[pl.ds(g * GK, GK)]], buf, sem)

    def out_desc(g, buf, sem):
        return pltpu.make_async_copy(
            buf, out_hbm.at[pl.ds(base + g * G, G)], sem)

    gather_desc(0, rows2.at[0], gsems[0]).start()

    def compute_group(g, rows, obuf):
        for r in range(G):
            o = g * G + r
            wrow = w_v[o]
            wj = [_bcast_lane(wrow, j) for j in range(K)]
            b16 = b_v[o // L]
            bias_b = _bcast_lane(b16, o % L)

            def chunk_body(ci, _):
                for cu in range(CUNROLL):
                    c = ci * CUNROLL + cu
                    sl = pl.ds(c * L, L)
                    accs = []
                    for ch in range(4):
                        a = wj[ch] * rows[r * K + ch, sl]
                        if ch == 0:
                            a = a + bias_b
                        for j in range(ch + 4, K, 4):
                            a = wj[j] * rows[r * K + j, sl] + a
                        accs.append(a)
                    obuf[r, sl] = (accs[0] + accs[1]) + (accs[2] + accs[3])
                return 0

            lax.fori_loop(0, CHUNKS // CUNROLL, chunk_body, 0, unroll=False)

    def pair_body(i, carry):
        for b in range(2):
            g = 2 * i + b
            nxt = g + 1

            @pl.when(nxt < NGRP)
            def _():
                gather_desc(nxt, rows2.at[1 - b], gsems[1 - b]).start()

            gather_desc(g, rows2.at[b], gsems[b]).wait()

            @pl.when(g >= 2)
            def _():
                out_desc(g - 2, obuf2.at[b], osems[b]).wait()

            compute_group(g, rows2.at[b], obuf2.at[b])
            out_desc(g, obuf2.at[b], osems[b]).start()
        return carry

    lax.fori_loop(0, NGRP // 2, pair_body, 0)

    out_desc(NGRP - 2, obuf2.at[0], osems[0]).wait()
    out_desc(NGRP - 1, obuf2.at[1], osems[1]).wait()


@functools.partial(
    pl.kernel,
    out_type=jax.ShapeDtypeStruct((OUT_F, T), jnp.float32),
    mesh=plsc.VectorSubcoreMesh(core_axis_name="c", subcore_axis_name="s"),
    scratch_types=[
        pltpu.VMEM((O_PER_W * K,), jnp.int32),
        pltpu.VMEM((O_PER_W, K), jnp.float32),
        pltpu.VMEM((O_PER_W // L, L), jnp.float32),
        pltpu.VMEM((2, GK, T), jnp.float32),
        pltpu.VMEM((2, G, T), jnp.float32),
        pltpu.SemaphoreType.DMA,
        pltpu.SemaphoreType.DMA,
        pltpu.SemaphoreType.DMA,
        pltpu.SemaphoreType.DMA,
    ],
)
def _sc_kernel(xT_hbm, idx_hbm, w_hbm, b_hbm, out_hbm,
               idx_v, w_v, b_v, rows2, obuf2,
               gsem0, gsem1, osem0, osem1):
    _sc_body(xT_hbm, idx_hbm, w_hbm, b_hbm, out_hbm,
             idx_v, w_v, b_v, rows2, obuf2,
             gsem0, gsem1, osem0, osem1)


def kernel(x, in_index_per_out, weight, bias):
    xT = x.T
    idx_flat = in_index_per_out.astype(jnp.int32).reshape(-1)
    b2 = bias.reshape(OUT_F // L, L)
    outT = _sc_kernel(xT, idx_flat, weight, b2)
    return outT.T


# speedup vs baseline: 1.1665x; 1.1665x over previous
import functools

import jax
import jax.numpy as jnp
from jax import lax
from jax.experimental import pallas as pl
from jax.experimental.pallas import tpu as pltpu
from jax.experimental.pallas import tpu_sc as plsc

OUT_F = 8192
IN_F = 8192
K = 16
T = 512
L = 16
NC = 2
NS = 16
NW = NC * NS
O_PER_W = OUT_F // NW
CHUNKS = T // L
G = 4
GK = G * K
NGRP = O_PER_W // G
CUNROLL = 2


def _bcast_lane(vec, lane):
    idx = jnp.broadcast_to(jnp.asarray(lane, jnp.int32), (L,))
    return vec.at[idx].get(mode="promise_in_bounds")


def _sc_body(xT_hbm, idx_hbm, w_hbm, b_hbm, out_hbm,
             idx_v, w_v, b_v, rows2, obuf2,
             gsem0, gsem1, osem0, osem1):
    wid = lax.axis_index("s") * NC + lax.axis_index("c")
    base = wid * O_PER_W

    pltpu.sync_copy(idx_hbm.at[pl.ds(base * K, O_PER_W * K)], idx_v)
    pltpu.sync_copy(w_hbm.at[pl.ds(base, O_PER_W)], w_v)
    pltpu.sync_copy(b_hbm.at[pl.ds(wid * (O_PER_W // L), O_PER_W // L)], b_v)

    gsems = (gsem0, gsem1)
    osems = (osem0, osem1)

    def gather_desc(g, buf, sem):
        return pltpu.make_async_copy(
            xT_hbm.at[idx_v.at[pl.ds(g * GK, GK)]], buf, sem)

    def out_desc(g, buf, sem):
        return pltpu.make_async_copy(
            buf, out_hbm.at[pl.ds(base + g * G, G)], sem)

    gather_desc(0, rows2.at[0], gsems[0]).start()

    def compute_group(g, rows, obuf):
        for r in range(G):
            o = g * G + r
            wrow = w_v[o]
            wj = [_bcast_lane(wrow, j) for j in range(K)]
            b16 = b_v[o // L]
            bias_b = _bcast_lane(b16, o % L)

            def chunk_body(ci, _):
                for cu in range(CUNROLL):
                    c = ci * CUNROLL + cu
                    sl = pl.ds(c * L, L)
                    accs = []
                    for ch in range(4):
                        a = wj[ch] * rows[r * K + ch, sl]
                        if ch == 0:
                            a = a + bias_b
                        for j in range(ch + 4, K, 4):
                            a = wj[j] * rows[r * K + j, sl] + a
                        accs.append(a)
                    obuf[r, sl] = (accs[0] + accs[1]) + (accs[2] + accs[3])
                return 0

            lax.fori_loop(0, CHUNKS // CUNROLL, chunk_body, 0, unroll=False)

    def pair_body(i, carry):
        for b in range(2):
            g = 2 * i + b
            nxt = g + 1

            @pl.when(nxt < NGRP)
            def _():
                gather_desc(nxt, rows2.at[1 - b], gsems[1 - b]).start()

            gather_desc(g, rows2.at[b], gsems[b]).wait()

            @pl.when(g >= 2)
            def _():
                out_desc(g - 2, obuf2.at[b], osems[b]).wait()

            compute_group(g, rows2.at[b], obuf2.at[b])
            out_desc(g, obuf2.at[b], osems[b]).start()
        return carry

    lax.fori_loop(0, NGRP // 2, pair_body, 0)

    out_desc(NGRP - 2, obuf2.at[0], osems[0]).wait()
    out_desc(NGRP - 1, obuf2.at[1], osems[1]).wait()


@functools.partial(
    pl.kernel,
    out_type=jax.ShapeDtypeStruct((OUT_F, T), jnp.float32),
    mesh=plsc.VectorSubcoreMesh(core_axis_name="c", subcore_axis_name="s"),
    scratch_types=[
        pltpu.VMEM((O_PER_W * K,), jnp.int32),
        pltpu.VMEM((O_PER_W, K), jnp.float32),
        pltpu.VMEM((O_PER_W // L, L), jnp.float32),
        pltpu.VMEM((2, GK, T), jnp.float32),
        pltpu.VMEM((2, G, T), jnp.float32),
        pltpu.SemaphoreType.DMA,
        pltpu.SemaphoreType.DMA,
        pltpu.SemaphoreType.DMA,
        pltpu.SemaphoreType.DMA,
    ],
)
def _sc_kernel(xT_hbm, idx_hbm, w_hbm, b_hbm, out_hbm,
               idx_v, w_v, b_v, rows2, obuf2,
               gsem0, gsem1, osem0, osem1):
    _sc_body(xT_hbm, idx_hbm, w_hbm, b_hbm, out_hbm,
             idx_v, w_v, b_v, rows2, obuf2,
             gsem0, gsem1, osem0, osem1)


def kernel(x, in_index_per_out, weight, bias):
    xT = x.T
    idx_flat = in_index_per_out.astype(jnp.int32).reshape(-1)
    b2 = bias.reshape(OUT_F // L, L)
    outT = _sc_kernel(xT, idx_flat, weight, b2)
    return outT.T


# speedup vs baseline: 1.1748x; 1.0070x over previous
import functools

import jax
import jax.numpy as jnp
from jax import lax
from jax.experimental import pallas as pl
from jax.experimental.pallas import tpu as pltpu
from jax.experimental.pallas import tpu_sc as plsc

OUT_F = 8192
IN_F = 8192
K = 16
T = 512
L = 16
NC = 2
NS = 16
NW = NC * NS
O_PER_W = OUT_F // NW
CHUNKS = T // L
G = 4
GK = G * K
NGRP = O_PER_W // G
CUNROLL = 1


def _bcast_lane(vec, lane):
    idx = jnp.broadcast_to(jnp.asarray(lane, jnp.int32), (L,))
    return vec.at[idx].get(mode="promise_in_bounds")


def _sc_body(xT_hbm, idx_hbm, w_hbm, b_hbm, out_hbm,
             idx_v, w_v, b_v, rows2, obuf2,
             gsem0, gsem1, osem0, osem1):
    wid = lax.axis_index("s") * NC + lax.axis_index("c")
    base = wid * O_PER_W

    pltpu.sync_copy(idx_hbm.at[pl.ds(base * K, O_PER_W * K)], idx_v)
    pltpu.sync_copy(w_hbm.at[pl.ds(base, O_PER_W)], w_v)
    pltpu.sync_copy(b_hbm.at[pl.ds(wid * (O_PER_W // L), O_PER_W // L)], b_v)

    gsems = (gsem0, gsem1)
    osems = (osem0, osem1)

    def gather_desc(g, buf, sem):
        return pltpu.make_async_copy(
            xT_hbm.at[idx_v.at[pl.ds(g * GK, GK)]], buf, sem)

    def out_desc(g, buf, sem):
        return pltpu.make_async_copy(
            buf, out_hbm.at[pl.ds(base + g * G, G)], sem)

    gather_desc(0, rows2.at[0], gsems[0]).start()

    def compute_group(g, rows, obuf):
        for r in range(G):
            o = g * G + r
            wrow = w_v[o]
            wj = [_bcast_lane(wrow, j) for j in range(K)]
            b16 = b_v[o // L]
            bias_b = _bcast_lane(b16, o % L)

            def chunk_body(ci, _):
                for cu in range(CUNROLL):
                    c = ci * CUNROLL + cu
                    sl = pl.ds(c * L, L)
                    accs = []
                    for ch in range(4):
                        a = wj[ch] * rows[r * K + ch, sl]
                        if ch == 0:
                            a = a + bias_b
                        for j in range(ch + 4, K, 4):
                            a = wj[j] * rows[r * K + j, sl] + a
                        accs.append(a)
                    obuf[r, sl] = (accs[0] + accs[1]) + (accs[2] + accs[3])
                return 0

            lax.fori_loop(0, CHUNKS // CUNROLL, chunk_body, 0, unroll=False)

    def pair_body(i, carry):
        for b in range(2):
            g = 2 * i + b
            nxt = g + 1

            @pl.when(nxt < NGRP)
            def _():
                gather_desc(nxt, rows2.at[1 - b], gsems[1 - b]).start()

            gather_desc(g, rows2.at[b], gsems[b]).wait()

            @pl.when(g >= 2)
            def _():
                out_desc(g - 2, obuf2.at[b], osems[b]).wait()

            compute_group(g, rows2.at[b], obuf2.at[b])
            out_desc(g, obuf2.at[b], osems[b]).start()
        return carry

    lax.fori_loop(0, NGRP // 2, pair_body, 0)

    out_desc(NGRP - 2, obuf2.at[0], osems[0]).wait()
    out_desc(NGRP - 1, obuf2.at[1], osems[1]).wait()


@functools.partial(
    pl.kernel,
    out_type=jax.ShapeDtypeStruct((OUT_F, T), jnp.float32),
    mesh=plsc.VectorSubcoreMesh(core_axis_name="c", subcore_axis_name="s"),
    scratch_types=[
        pltpu.VMEM((O_PER_W * K,), jnp.int32),
        pltpu.VMEM((O_PER_W, K), jnp.float32),
        pltpu.VMEM((O_PER_W // L, L), jnp.float32),
        pltpu.VMEM((2, GK, T), jnp.float32),
        pltpu.VMEM((2, G, T), jnp.float32),
        pltpu.SemaphoreType.DMA,
        pltpu.SemaphoreType.DMA,
        pltpu.SemaphoreType.DMA,
        pltpu.SemaphoreType.DMA,
    ],
)
def _sc_kernel(xT_hbm, idx_hbm, w_hbm, b_hbm, out_hbm,
               idx_v, w_v, b_v, rows2, obuf2,
               gsem0, gsem1, osem0, osem1):
    _sc_body(xT_hbm, idx_hbm, w_hbm, b_hbm, out_hbm,
             idx_v, w_v, b_v, rows2, obuf2,
             gsem0, gsem1, osem0, osem1)


def kernel(x, in_index_per_out, weight, bias):
    xT = x.T
    idx_flat = in_index_per_out.astype(jnp.int32).reshape(-1)
    b2 = bias.reshape(OUT_F // L, L)
    outT = _sc_kernel(xT, idx_flat, weight, b2)
    return outT.T


# speedup vs baseline: 1.3464x; 1.1461x over previous
import functools

import jax
import jax.numpy as jnp
from jax import lax
from jax.experimental import pallas as pl
from jax.experimental.pallas import tpu as pltpu
from jax.experimental.pallas import tpu_sc as plsc

OUT_F = 8192
IN_F = 8192
K = 16
T = 512
L = 16
NC = 2
NS = 16
NW = NC * NS
O_PER_W = OUT_F // NW
CHUNKS = T // L
G = 4
GK = G * K
NGRP = O_PER_W // G
TB = 8


def _bcast_lane(vec, lane):
    idx = jnp.broadcast_to(jnp.asarray(lane, jnp.int32), (L,))
    return vec.at[idx].get(mode="promise_in_bounds")


def _sc_body(xT_hbm, idx_hbm, w_hbm, b_hbm, out_hbm,
             idx_v, w_v, b_v, rows2, obuf2,
             gsem0, gsem1, osem0, osem1):
    wid = lax.axis_index("s") * NC + lax.axis_index("c")
    base = wid * O_PER_W

    pltpu.sync_copy(idx_hbm.at[pl.ds(base * K, O_PER_W * K)], idx_v)
    pltpu.sync_copy(w_hbm.at[pl.ds(base, O_PER_W)], w_v)
    pltpu.sync_copy(b_hbm.at[pl.ds(wid * (O_PER_W // L), O_PER_W // L)], b_v)

    gsems = (gsem0, gsem1)
    osems = (osem0, osem1)

    def gather_desc(g, buf, sem):
        return pltpu.make_async_copy(
            xT_hbm.at[idx_v.at[pl.ds(g * GK, GK)]], buf, sem)

    def out_desc(g, buf, sem):
        return pltpu.make_async_copy(
            buf, out_hbm.at[pl.ds(base + g * G, G)], sem)

    gather_desc(0, rows2.at[0], gsems[0]).start()

    def compute_group(g, rows, obuf):
        for r in range(G):
            o = g * G + r
            wrow = w_v[o]
            b16 = b_v[o // L]
            bias_b = _bcast_lane(b16, o % L)

            def block_body(tb, _):
                base_c = tb * TB
                accs = [bias_b] * TB
                for j in range(K):
                    wj = _bcast_lane(wrow, j)
                    for c in range(TB):
                        sl = pl.ds((base_c + c) * L, L)
                        accs[c] = wj * rows[r * K + j, sl] + accs[c]
                for c in range(TB):
                    obuf[r, pl.ds((base_c + c) * L, L)] = accs[c]
                return 0

            lax.fori_loop(0, CHUNKS // TB, block_body, 0, unroll=False)

    def pair_body(i, carry):
        for b in range(2):
            g = 2 * i + b
            nxt = g + 1

            @pl.when(nxt < NGRP)
            def _():
                gather_desc(nxt, rows2.at[1 - b], gsems[1 - b]).start()

            gather_desc(g, rows2.at[b], gsems[b]).wait()

            @pl.when(g >= 2)
            def _():
                out_desc(g - 2, obuf2.at[b], osems[b]).wait()

            compute_group(g, rows2.at[b], obuf2.at[b])
            out_desc(g, obuf2.at[b], osems[b]).start()
        return carry

    lax.fori_loop(0, NGRP // 2, pair_body, 0)

    out_desc(NGRP - 2, obuf2.at[0], osems[0]).wait()
    out_desc(NGRP - 1, obuf2.at[1], osems[1]).wait()


@functools.partial(
    pl.kernel,
    out_type=jax.ShapeDtypeStruct((OUT_F, T), jnp.float32),
    mesh=plsc.VectorSubcoreMesh(core_axis_name="c", subcore_axis_name="s"),
    scratch_types=[
        pltpu.VMEM((O_PER_W * K,), jnp.int32),
        pltpu.VMEM((O_PER_W, K), jnp.float32),
        pltpu.VMEM((O_PER_W // L, L), jnp.float32),
        pltpu.VMEM((2, GK, T), jnp.float32),
        pltpu.VMEM((2, G, T), jnp.float32),
        pltpu.SemaphoreType.DMA,
        pltpu.SemaphoreType.DMA,
        pltpu.SemaphoreType.DMA,
        pltpu.SemaphoreType.DMA,
    ],
)
def _sc_kernel(xT_hbm, idx_hbm, w_hbm, b_hbm, out_hbm,
               idx_v, w_v, b_v, rows2, obuf2,
               gsem0, gsem1, osem0, osem1):
    _sc_body(xT_hbm, idx_hbm, w_hbm, b_hbm, out_hbm,
             idx_v, w_v, b_v, rows2, obuf2,
             gsem0, gsem1, osem0, osem1)


def kernel(x, in_index_per_out, weight, bias):
    xT = x.T
    idx_flat = in_index_per_out.astype(jnp.int32).reshape(-1)
    b2 = bias.reshape(OUT_F // L, L)
    outT = _sc_kernel(xT, idx_flat, weight, b2)
    return outT.T
